# Initial kernel scaffold; baseline (speedup 1.0000x reference)
import jax
import jax.numpy as jnp
from jax.experimental import pallas as pl


def kernel(x, coords):
    raise NotImplementedError("write your pallas kernel here")


# speedup vs baseline: 2.1326x; 2.1326x over previous
import functools
import math

import jax
import jax.numpy as jnp
from jax import lax
from jax.experimental import pallas as pl
from jax.experimental.pallas import tpu as pltpu
from jax.experimental.pallas import tpu_sc as plsc

_NH = 5
_NRES = 90
_S = 0.5
_B = 4
_N = 8192
_C = 64
_NX = 384
_NY = 384

_NC = 2
_NS = 16
_NW = _NC * _NS
_NPTS = _B * _N
_PTS_PER_TILE = _NPTS // _NW
_PTS_PER_CHUNK = 4
_TAPS = _NH * _NH
_IDX_PER_CHUNK = _PTS_PER_CHUNK * _TAPS
_CHUNKS_PER_GRP = 64
_GRPS_PER_TILE = _PTS_PER_TILE // (_PTS_PER_CHUNK * _CHUNKS_PER_GRP)
_NGRP = _NW * _GRPS_PER_TILE
_GRP_PTS = _PTS_PER_CHUNK * _CHUNKS_PER_GRP


def _prep_body(coords_ref, idx_ref, w_ref):
    inv_norm = 1.0 / (_S * math.sqrt(2.0 * math.pi))
    nh_m = (_NH - 1) / 2 + 0.5

    posy = coords_ref[0, 0:1, :] * (_NY - 1)
    posx = coords_ref[0, 1:2, :] * (_NX - 1)
    rpx = jnp.round(posx)
    rpy = jnp.round(posy)

    i90 = lax.broadcasted_iota(jnp.int32, (_NRES, 1), 0).astype(jnp.float32)
    off_n = (nh_m + i90 * (-2.0 * nh_m / (_NRES - 1))).astype(jnp.float32)

    pxo = jnp.clip(rpx - off_n, 0.0, float(_NX))
    pyo = jnp.clip(rpy - off_n, 0.0, float(_NX))
    wx = jnp.exp(-0.5 * ((pxo - posx) / _S) ** 2) * inv_norm
    wy = jnp.exp(-0.5 * ((pyo - posy) / _S) ** 2) * inv_norm
    wx5 = wx.reshape(_NH, _NRES // _NH, _N).sum(axis=1)
    wy5 = wy.reshape(_NH, _NRES // _NH, _N).sum(axis=1)

    w2 = (wx5[:, None, :] * wy5[None, :, :]).reshape(_TAPS, _N)
    den = w2.sum(axis=0, keepdims=True)
    w_ref[0] = w2 / den

    i5 = lax.broadcasted_iota(jnp.int32, (_NH, 1), 0).astype(jnp.float32)
    off_i = i5 - ((_NH - 1) // 2)
    pxi = jnp.clip(jnp.round(rpx - off_i), 0.0, float(_NX - 1)).astype(jnp.int32)
    pyi = jnp.clip(jnp.round(rpy - off_i), 0.0, float(_NX - 1)).astype(jnp.int32)
    boff = pl.program_id(0) * (_NX * _NY)
    idx25 = (pxi[:, None, :] * _NY + pyi[None, :, :]).reshape(_TAPS, _N) + boff
    idx_ref[0] = idx25


def _prep(coords_t):
    return pl.pallas_call(
        _prep_body,
        grid=(_B,),
        in_specs=[pl.BlockSpec((1, 2, _N), lambda i: (i, 0, 0))],
        out_specs=[
            pl.BlockSpec((1, _TAPS, _N), lambda i: (i, 0, 0)),
            pl.BlockSpec((1, _TAPS, _N), lambda i: (i, 0, 0)),
        ],
        out_shape=[
            jax.ShapeDtypeStruct((_B, _TAPS, _N), jnp.int32),
            jax.ShapeDtypeStruct((_B, _TAPS, _N), jnp.float32),
        ],
    )(coords_t)


def _splat_lane(vec, lane):
    idx = jnp.full((16, 1), lane, dtype=jnp.int32)
    dn = lax.GatherDimensionNumbers(
        offset_dims=(), collapsed_slice_dims=(0,), start_index_map=(0,))
    return lax.gather(vec, idx, dn, (1,),
                      mode=lax.GatherScatterMode.PROMISE_IN_BOUNDS)


def _sc_body(x_hbm, idx_hbm, w_hbm, out_hbm, idx_v, w_v, rows_v, out_v,
             sem_a, sem_b):
    wid = lax.axis_index("s") * _NC + lax.axis_index("c")
    sems = (sem_a, sem_b)

    def gather(ci, slot):
        return pltpu.make_async_copy(
            x_hbm.at[idx_v.at[ci]], rows_v.at[slot], sems[slot])

    def compute(ci, slot):
        wvecs = [w_v[ci, pl.ds(o, 16)] for o in (0, 16, 32, 48, 64, 80, 84)]
        for p in range(_PTS_PER_CHUNK):
            acc = [jnp.zeros((16,), jnp.float32) for _ in range(_C // 16)]
            for k in range(_TAPS):
                off = p * _TAPS + k
                r, lane = (off // 16, off % 16) if off < 96 else (6, off - 84)
                wsp = _splat_lane(wvecs[r], lane)
                for h in range(_C // 16):
                    acc[h] = acc[h] + wsp * rows_v[slot, off, pl.ds(h * 16, 16)]
            for h in range(_C // 16):
                out_v[ci * _PTS_PER_CHUNK + p, pl.ds(h * 16, 16)] = acc[h]

    for g in range(_GRPS_PER_TILE):
        a = wid * _GRPS_PER_TILE + g
        pltpu.sync_copy(idx_hbm.at[a], idx_v)
        pltpu.sync_copy(w_hbm.at[a], w_v)

        gather(0, 0).start()

        def body2(it, carry):
            for b2 in range(2):
                ci = it * 2 + b2
                nxt = ci + 1

                @pl.when(nxt < _CHUNKS_PER_GRP)
                def _():
                    gather(nxt, 1 - b2).start()

                gather(ci, b2).wait()
                compute(ci, b2)
            return carry

        lax.fori_loop(0, _CHUNKS_PER_GRP // 2, body2, 0)
        pltpu.sync_copy(out_v, out_hbm.at[pl.ds(a * _GRP_PTS, _GRP_PTS)])


@functools.partial(jax.jit, static_argnums=())
def _sc_gather(x_flat, idxr, wr):
    mesh = plsc.VectorSubcoreMesh(core_axis_name="c", subcore_axis_name="s")
    f = functools.partial(
        pl.kernel,
        mesh=mesh,
        out_type=jax.ShapeDtypeStruct((_NPTS, _C), jnp.float32),
        scratch_types=[
            pltpu.VMEM((_CHUNKS_PER_GRP, _IDX_PER_CHUNK), jnp.int32),
            pltpu.VMEM((_CHUNKS_PER_GRP, _IDX_PER_CHUNK), jnp.float32),
            pltpu.VMEM((2, _IDX_PER_CHUNK, _C), jnp.float32),
            pltpu.VMEM((_GRP_PTS, _C), jnp.float32),
            pltpu.SemaphoreType.DMA,
            pltpu.SemaphoreType.DMA,
        ],
        compiler_params=pltpu.CompilerParams(use_tc_tiling_on_sc=False),
    )(_sc_body)
    return f(x_flat, idxr, wr)


def kernel(x, coords):
    b, c, nx, ny = x.shape
    x_flat = x.transpose(0, 2, 3, 1).reshape(b * nx * ny, c)
    idx_t, w_t = _prep(coords.transpose(0, 2, 1))
    idxr = idx_t.transpose(0, 2, 1).reshape(_NGRP, _CHUNKS_PER_GRP, _IDX_PER_CHUNK)
    wr = w_t.transpose(0, 2, 1).reshape(_NGRP, _CHUNKS_PER_GRP, _IDX_PER_CHUNK)
    out_rows = _sc_gather(x_flat, idxr, wr)
    return out_rows.reshape(b, _N, c).transpose(0, 2, 1)
